# baseline (speedup 1.0000x reference)
import functools

import jax
import jax.numpy as jnp
from jax import lax
from jax.experimental import pallas as pl
from jax.experimental.pallas import tpu as pltpu
from jax.experimental.pallas import tpu_sc as plsc

_BR = 128
_NW = 32
_NBUF = 4


def _rowsum_kernel(x_ref, out_ref):
    out_ref[...] = jnp.sum(x_ref[...], axis=1, keepdims=True)


def _median_kernel(ctc_ref, part_ref, after_ref, call_ref):
    r_tc = ctc_ref.shape[0]
    ctc = ctc_ref[...]
    csc = jnp.sum(part_ref[...], axis=1, keepdims=True)
    call_ref[:r_tc, :] = ctc
    call_ref[r_tc:, :] = csc

    b1 = jax.lax.bitcast_convert_type(ctc, jnp.int32)
    b2 = jax.lax.bitcast_convert_type(csc, jnp.int32)
    pos1 = b1 > 0
    pos2 = b2 > 0
    n_pos = jnp.sum(pos1.astype(jnp.int32)) + jnp.sum(pos2.astype(jnp.int32))
    target = (n_pos - 1) // 2 + 1

    def body(i, lo):
        cand = lo + (jnp.int32(1) << (30 - i))
        g = (jnp.sum((pos1 & (b1 < cand)).astype(jnp.int32))
             + jnp.sum((pos2 & (b2 < cand)).astype(jnp.int32)))
        return jnp.where(g >= target, lo, cand)

    ans = jax.lax.fori_loop(0, 31, body, jnp.int32(0))
    after = jax.lax.bitcast_convert_type(ans, jnp.float32)
    after = jnp.where(n_pos == 0, jnp.inf, after)
    after_ref[...] = jnp.full(after_ref.shape, after, dtype=after_ref.dtype)


def _finalize_kernel(x_ref, c_ref, after_ref, out_ref):
    g = x_ref.shape[1]
    c = c_ref[...]
    c_adj = c + (c == 0.0).astype(c.dtype)
    scale = after_ref[0, 0] / c_adj
    out_ref[:, :g] = jnp.log1p(x_ref[...] * scale)
    t = jnp.log10(c_adj)
    out_ref[:, g:] = jnp.broadcast_to(t, (t.shape[0], 2))


def _make_sc_rowsum(g, row0, n_sc):
    rpw = n_sc // _NW

    def body(x_hbm, out_hbm, buf, cv, *sems):
        cid = lax.axis_index("c")
        sid = lax.axis_index("s")
        wid = sid * 2 + cid
        base = row0 + wid * rpw

        def chunk_copy(k, b):
            return pltpu.make_async_copy(
                x_hbm.at[pl.ds(base + k, 1), :], buf.at[b], sems[b])

        for b in range(_NBUF):
            chunk_copy(b, b).start()

        zero = jnp.zeros((16,), jnp.float32)
        unroll = 28
        niter = (g // 16) // unroll

        def quad_body(p, _):
            for b in range(_NBUF):
                k = p * _NBUF + b
                chunk_copy(k, b).wait()

                def inner(j, accs):
                    a = list(accs)
                    off = j * (unroll * 16)
                    for u in range(unroll):
                        a[u % 4] = a[u % 4] + buf[b, 0, pl.ds(off + u * 16, 16)]
                    return tuple(a)

                a = lax.fori_loop(0, niter, inner, (zero,) * 4)
                cv[pl.ds(k * 16, 16)] = (a[0] + a[1]) + (a[2] + a[3])

                @pl.when(k + _NBUF < rpw)
                def _():
                    chunk_copy(k + _NBUF, b).start()
            return 0

        lax.fori_loop(0, rpw // _NBUF, quad_body, 0)
        pltpu.sync_copy(cv, out_hbm.at[pl.ds(wid * rpw * 16, rpw * 16)])

    mesh = plsc.VectorSubcoreMesh(core_axis_name="c", subcore_axis_name="s")
    return functools.partial(
        pl.kernel, body, mesh=mesh,
        out_type=jax.ShapeDtypeStruct((n_sc * 16,), jnp.float32),
        scratch_types=[
            pltpu.VMEM((_NBUF, 1, g), jnp.float32),
            pltpu.VMEM((rpw * 16,), jnp.float32),
        ] + [pltpu.SemaphoreType.DMA] * _NBUF,
    )()


def kernel(X):
    X = jnp.squeeze(X)
    n, g = X.shape
    r_tc = n // 2
    n_sc = n - r_tc

    counts_tc = pl.pallas_call(
        _rowsum_kernel,
        grid=(r_tc // _BR,),
        in_specs=[pl.BlockSpec((_BR, g), lambda i: (i, 0))],
        out_specs=pl.BlockSpec((_BR, 1), lambda i: (i, 0)),
        out_shape=jax.ShapeDtypeStruct((r_tc, 1), X.dtype),
    )(X)

    part_sc = _make_sc_rowsum(g, r_tc, n_sc)(X).reshape(n_sc, 16)

    after, counts = pl.pallas_call(
        _median_kernel,
        out_shape=(
            jax.ShapeDtypeStruct((1, 1), X.dtype),
            jax.ShapeDtypeStruct((n, 1), X.dtype),
        ),
    )(counts_tc, part_sc)

    out = pl.pallas_call(
        _finalize_kernel,
        grid=(n // _BR,),
        in_specs=[
            pl.BlockSpec((_BR, g), lambda i: (i, 0)),
            pl.BlockSpec((_BR, 1), lambda i: (i, 0)),
            pl.BlockSpec((1, 1), lambda i: (0, 0)),
        ],
        out_specs=pl.BlockSpec((_BR, g + 2), lambda i: (i, 0)),
        out_shape=jax.ShapeDtypeStruct((n, g + 2), X.dtype),
    )(X, counts, after)
    return out
